# baseline (speedup 1.0000x reference)
import functools

import numpy as np
import jax
import jax.numpy as jnp
from jax.experimental import pallas as pl
from jax.experimental.pallas import tpu as pltpu

_NT = 4096
_NSEG = 512
_NB = 4
_D = 1024


def _seg_idx() -> np.ndarray:
    t = np.linspace(1, _NT, _NSEG + 1)
    return np.asarray([int(round(x)) - 1 for x in t][:-1], dtype=np.int32)


_IDX = _seg_idx()


def _copy_body(idx_ref, x_ref, o_ref):
    o_ref[...] = x_ref[...]


@jax.jit
def _tc_gather(inp, idx):
    x = inp.reshape(_NB, _NT, 8, _D // 8)
    grid_spec = pltpu.PrefetchScalarGridSpec(
        num_scalar_prefetch=1,
        grid=(_NB, _NSEG),
        in_specs=[
            pl.BlockSpec((1, 1, 8, _D // 8),
                         lambda b, s, idx_ref: (b, idx_ref[s], 0, 0)),
        ],
        out_specs=pl.BlockSpec((1, 1, 8, _D // 8),
                               lambda b, s, idx_ref: (b, s, 0, 0)),
    )
    out = pl.pallas_call(
        _copy_body,
        grid_spec=grid_spec,
        out_shape=jax.ShapeDtypeStruct((_NB, _NSEG, 8, _D // 8), jnp.float32),
    )(idx, x)
    return out.reshape(_NB, _NSEG, _D)


def kernel(inp, n_segments):
    del n_segments
    return _tc_gather(inp, jnp.asarray(_IDX))


# speedup vs baseline: 38.3798x; 38.3798x over previous
import functools

import jax
import jax.numpy as jnp
from jax import lax
from jax.experimental import pallas as pl
from jax.experimental.pallas import tpu as pltpu
from jax.experimental.pallas import tpu_sc as plsc

_NT = 4096
_NSEG = 512
_NB = 4
_D = 1024

_NC = 2
_NS = 16
_NW = _NC * _NS
_LANES = 16

_B_TOTAL = _NB * _NSEG
_B_PER_W = _B_TOTAL // _NW

_mesh = plsc.VectorSubcoreMesh(core_axis_name="c", subcore_axis_name="s")


@functools.partial(
    pl.kernel,
    mesh=_mesh,
    out_type=jax.ShapeDtypeStruct((_B_TOTAL, _D), jnp.float32),
    scratch_types=[
        pltpu.VMEM((_B_PER_W,), jnp.int32),
        pltpu.VMEM((_B_PER_W, _D), jnp.float32),
        pltpu.SemaphoreType.DMA,
    ],
)
def _gather_rows(table_hbm, out_hbm, idx_v, rows_v, sem):
    wid = lax.axis_index("s") * _NC + lax.axis_index("c")
    base = wid * _B_PER_W
    for c in range(_B_PER_W // _LANES):
        r = base + c * _LANES + lax.iota(jnp.int32, _LANES)
        b = lax.shift_right_logical(r, 9)
        s = lax.bitwise_and(r, 511)
        t_raw = lax.shift_right_logical(s * 4095 + 256, 9)
        t = t_raw - jnp.where(s == 256, 1, 0).astype(jnp.int32)
        idx_v[pl.ds(c * _LANES, _LANES)] = lax.shift_left(b, 12) + t
    pltpu.async_copy(table_hbm.at[idx_v], rows_v, sem).wait()
    pltpu.sync_copy(rows_v, out_hbm.at[pl.ds(base, _B_PER_W)])


def kernel(inp, n_segments):
    del n_segments
    nb, nt, d = inp.shape
    table = inp.reshape(nb * nt, d)
    out = _gather_rows(table)
    return out.reshape(nb, _NSEG, d)
